# baseline (speedup 1.0000x reference)
import functools

import jax
import jax.numpy as jnp
from jax import lax
from jax.experimental import pallas as pl
from jax.experimental.pallas import tpu as pltpu
from jax.experimental.pallas import tpu_sc as plsc

_E = 64
_TOPK = 8
_ALPHA = 0.01
_LANES = 16
_NSUB = 16


def _router_body(x_ref, w_ref, idx_ref, wgt_ref, ssum_ref, ssum_acc,
                 *, bt, seq_len, bsz, nblocks):
    i = pl.program_id(0)
    x = x_ref[...]
    w = w_ref[...]

    logits = lax.dot_general(
        w, x, dimension_numbers=(((1,), (1,)), ((), ())),
        preferred_element_type=jnp.float32)

    m = jnp.max(logits, axis=0, keepdims=True)
    e = jnp.exp(logits - m)
    scores = e / jnp.sum(e, axis=0, keepdims=True)

    ids = lax.broadcasted_iota(jnp.int32, (_E, bt), 0)
    work = scores
    vals = []
    idxs = []
    for _ in range(_TOPK):
        cur = jnp.max(work, axis=0, keepdims=True)
        sel_idx = jnp.min(jnp.where(work == cur, ids, _E), axis=0,
                          keepdims=True)
        vals.append(cur)
        idxs.append(sel_idx)
        work = jnp.where(ids == sel_idx, -jnp.inf, work)

    topk_w = jnp.concatenate(vals, axis=0)
    topk_i = jnp.concatenate(idxs, axis=0)
    denom = jnp.sum(topk_w, axis=0, keepdims=True) + 1e-20
    idx_ref[...] = topk_i
    wgt_ref[...] = topk_w / denom

    ssum_e = jnp.sum(scores, axis=1, keepdims=True)
    blocks_per_batch = seq_len // bt
    b = i // blocks_per_batch
    onehot_b = (lax.broadcasted_iota(jnp.int32, (1, bsz), 1) == b
                ).astype(jnp.float32)

    @pl.when(i == 0)
    def _init():
        ssum_acc[...] = ssum_e * onehot_b

    @pl.when(i > 0)
    def _accum():
        ssum_acc[...] += ssum_e * onehot_b

    @pl.when(i == nblocks - 1)
    def _finalize():
        ssum_ref[...] = ssum_acc[...]


@functools.partial(jax.jit, static_argnames=("bt", "bsz"))
def _router(hidden_flat, weight, bt, bsz):
    n, d = hidden_flat.shape
    seq_len = n // bsz
    nblocks = n // bt

    body = functools.partial(_router_body, bt=bt, seq_len=seq_len, bsz=bsz,
                             nblocks=nblocks)
    return pl.pallas_call(
        body,
        grid=(nblocks,),
        in_specs=[
            pl.BlockSpec((bt, d), lambda i: (i, 0)),
            pl.BlockSpec((_E, d), lambda i: (0, 0)),
        ],
        out_specs=[
            pl.BlockSpec((_TOPK, bt), lambda i: (0, i)),
            pl.BlockSpec((_TOPK, bt), lambda i: (0, i)),
            pl.BlockSpec((_E, bsz), lambda i: (0, 0)),
        ],
        out_shape=[
            jax.ShapeDtypeStruct((_TOPK, n), jnp.int32),
            jax.ShapeDtypeStruct((_TOPK, n), jnp.float32),
            jax.ShapeDtypeStruct((_E, bsz), jnp.float32),
        ],
        scratch_shapes=[
            pltpu.VMEM((_E, bsz), jnp.float32),
        ],
        compiler_params=pltpu.CompilerParams(
            dimension_semantics=("arbitrary",)),
    )(hidden_flat, weight)


def _sc_aux_body(idx_hbm, ssum_hbm, out_hbm, idx_v, bins_v, ssum_v, part_v,
                 fin_v, shared_v, *, n, seq_len, bsz, scale):
    cid = lax.axis_index("c")
    sid = lax.axis_index("s")
    per_sub = (_TOPK * n) // _NSUB
    halves = per_sub // seq_len
    iters = seq_len // _LANES

    @pl.when(cid == 0)
    def _work():
        lanes = lax.broadcasted_iota(jnp.int32, (_LANES,), 0)
        ones = jnp.ones((_LANES,), jnp.int32)
        base = sid * per_sub
        pltpu.sync_copy(idx_hbm.at[pl.ds(base, per_sub)], idx_v)
        pltpu.sync_copy(ssum_hbm, ssum_v)

        total = jnp.zeros((_LANES,), jnp.float32)
        for half in range(halves):
            for c in range(_E):
                bins_v[pl.ds(c * _LANES, _LANES)] = jnp.zeros(
                    (_LANES,), jnp.int32)
            off = half * seq_len

            def hist_body(i, carry):
                v = idx_v[pl.ds(off + i * _LANES, _LANES)]
                plsc.addupdate_scatter(bins_v, [lanes * _E + v], ones)
                return carry

            lax.fori_loop(0, iters, hist_body, 0)

            b = ((sid * per_sub) % n + half * seq_len) // seq_len
            for g in range(_E // _LANES):
                cnt = bins_v[pl.ds(g * _LANES, _LANES)]
                for r in range(1, _LANES):
                    cnt = cnt + bins_v[pl.ds(r * _E + g * _LANES, _LANES)]
                sidx = (g * _LANES + lanes) * bsz + b
                scol = plsc.load_gather(ssum_v, [sidx])
                total = total + cnt.astype(jnp.float32) * scol

        part_v[...] = total
        pltpu.sync_copy(part_v, shared_v.at[pl.ds(sid * _LANES, _LANES)])
        plsc.subcore_barrier()

        @pl.when(sid == 0)
        def _fin():
            pltpu.sync_copy(shared_v, fin_v)
            acc = jnp.zeros((_LANES,), jnp.float32)
            for r in range(_NSUB):
                acc = acc + fin_v[pl.ds(r * _LANES, _LANES)]
            aux = jnp.sum(acc) * scale
            part_v[...] = aux * jnp.ones((_LANES,), jnp.float32)
            pltpu.sync_copy(part_v, out_hbm)


@functools.partial(jax.jit, static_argnames=("n", "seq_len", "bsz", "scale"))
def _sc_aux(idx_flat, ssum_flat, n, seq_len, bsz, scale):
    per_sub = (_TOPK * n) // _NSUB
    body = functools.partial(_sc_aux_body, n=n, seq_len=seq_len, bsz=bsz,
                             scale=scale)
    mesh = plsc.VectorSubcoreMesh(core_axis_name="c", subcore_axis_name="s")
    f = pl.kernel(
        body,
        out_type=jax.ShapeDtypeStruct((_LANES,), jnp.float32),
        mesh=mesh,
        scratch_types=[
            pltpu.VMEM((per_sub,), jnp.int32),
            pltpu.VMEM((_NSUB * _E,), jnp.int32),
            pltpu.VMEM((_E * bsz,), jnp.float32),
            pltpu.VMEM((_LANES,), jnp.float32),
            pltpu.VMEM((_NSUB * _LANES,), jnp.float32),
            pltpu.VMEM_SHARED((_NSUB * _LANES,), jnp.float32),
        ],
        compiler_params=pltpu.CompilerParams(needs_layout_passes=False),
    )
    return f(idx_flat, ssum_flat)


def kernel(hidden_states, weight):
    bsz, seq_len, d = hidden_states.shape
    hidden_flat = hidden_states.reshape(-1, d)
    n = bsz * seq_len
    scale = _ALPHA * (_E / (seq_len * _TOPK)) / (seq_len * bsz)

    nchunks = 2
    bsz_c = bsz // nchunks
    n_c = n // nchunks
    idx_parts, wgt_parts, aux_parts = [], [], []
    for c in range(nchunks):
        xc = lax.slice_in_dim(hidden_flat, c * n_c, (c + 1) * n_c, axis=0)
        idx_c, wgt_c, ssum_c = _router(xc, weight, bt=2048, bsz=bsz_c)
        aux_c = _sc_aux(idx_c.reshape(-1), ssum_c.reshape(-1),
                        n=n_c, seq_len=seq_len, bsz=bsz_c, scale=scale)
        idx_parts.append(idx_c)
        wgt_parts.append(wgt_c)
        aux_parts.append(aux_c[0])
    topk_i_t = jnp.concatenate(idx_parts, axis=1)
    topk_w_t = jnp.concatenate(wgt_parts, axis=1)
    return topk_i_t.T, topk_w_t.T, aux_parts[0] + aux_parts[1]


# speedup vs baseline: 3.6265x; 3.6265x over previous
import functools

import jax
import jax.numpy as jnp
from jax import lax
from jax.experimental import pallas as pl
from jax.experimental.pallas import tpu as pltpu

_E = 64
_TOPK = 8
_ALPHA = 0.01


def _router_body(x_ref, w_ref, idx_ref, wgt_ref, aux_ref, cnt_acc, ssum_acc,
                 *, bt, seq_len, bsz, nblocks):
    i = pl.program_id(0)
    x = x_ref[...]
    w = w_ref[...]

    logits = lax.dot_general(
        w, x, dimension_numbers=(((1,), (1,)), ((), ())),
        preferred_element_type=jnp.float32)

    m = jnp.max(logits, axis=0, keepdims=True)
    e = jnp.exp(logits - m)
    scores = e / jnp.sum(e, axis=0, keepdims=True)

    ids = lax.broadcasted_iota(jnp.int32, (_E, bt), 0)
    work = scores
    vals = []
    idxs = []
    for _ in range(_TOPK):
        cur = jnp.max(work, axis=0, keepdims=True)
        sel_idx = jnp.min(jnp.where(work == cur, ids, _E), axis=0,
                          keepdims=True)
        vals.append(cur)
        idxs.append(sel_idx)
        work = jnp.where(ids == sel_idx, -jnp.inf, work)

    topk_w = jnp.concatenate(vals, axis=0)
    topk_i = jnp.concatenate(idxs, axis=0)
    denom = jnp.sum(topk_w, axis=0, keepdims=True) + 1e-20
    idx_ref[...] = topk_i
    wgt_ref[...] = topk_w / denom

    sel_mask = (work == -jnp.inf).astype(jnp.float32)
    cnt_e = jnp.sum(sel_mask, axis=1, keepdims=True)
    ssum_e = jnp.sum(scores, axis=1, keepdims=True)

    blocks_per_batch = seq_len // bt
    b = i // blocks_per_batch
    onehot_b = (lax.broadcasted_iota(jnp.int32, (1, bsz), 1) == b
                ).astype(jnp.float32)

    @pl.when(i == 0)
    def _init():
        cnt_acc[...] = cnt_e * onehot_b
        ssum_acc[...] = ssum_e * onehot_b

    @pl.when(i > 0)
    def _accum():
        cnt_acc[...] += cnt_e * onehot_b
        ssum_acc[...] += ssum_e * onehot_b

    @pl.when(i == nblocks - 1)
    def _finalize():
        scale = _ALPHA * (_E / (seq_len * _TOPK)) / (seq_len * bsz)
        aux_ref[...] = (jnp.sum(cnt_acc[...] * ssum_acc[...]) * scale
                        ).reshape(1, 1)


@functools.partial(jax.jit, static_argnames=("bt", "bsz"))
def _router(hidden_flat, weight, bt, bsz):
    n, d = hidden_flat.shape
    seq_len = n // bsz
    nblocks = n // bt

    body = functools.partial(_router_body, bt=bt, seq_len=seq_len, bsz=bsz,
                             nblocks=nblocks)
    topk_i, topk_w, aux = pl.pallas_call(
        body,
        grid=(nblocks,),
        in_specs=[
            pl.BlockSpec((bt, d), lambda i: (i, 0)),
            pl.BlockSpec((_E, d), lambda i: (0, 0)),
        ],
        out_specs=[
            pl.BlockSpec((_TOPK, bt), lambda i: (0, i)),
            pl.BlockSpec((_TOPK, bt), lambda i: (0, i)),
            pl.BlockSpec((1, 1), lambda i: (0, 0)),
        ],
        out_shape=[
            jax.ShapeDtypeStruct((_TOPK, n), jnp.int32),
            jax.ShapeDtypeStruct((_TOPK, n), jnp.float32),
            jax.ShapeDtypeStruct((1, 1), jnp.float32),
        ],
        scratch_shapes=[
            pltpu.VMEM((_E, bsz), jnp.float32),
            pltpu.VMEM((_E, bsz), jnp.float32),
        ],
        compiler_params=pltpu.CompilerParams(
            dimension_semantics=("arbitrary",)),
    )(hidden_flat, weight)
    return topk_i, topk_w, aux[0, 0]


def kernel(hidden_states, weight):
    bsz, seq_len, d = hidden_states.shape
    hidden_flat = hidden_states.reshape(-1, d)
    topk_i_t, topk_w_t, aux = _router(hidden_flat, weight, bt=2048, bsz=bsz)
    return topk_i_t.T, topk_w_t.T, aux
